# baseline (speedup 1.0000x reference)
import functools

import jax
import jax.numpy as jnp
from jax import lax
from jax.experimental import pallas as pl
from jax.experimental.pallas import tpu as pltpu
from jax.experimental.pallas import tpu_sc as plsc

_B, _C, _H, _W, _K = 64, 8, 112, 200, 128
_HW = _H * _W
_N = _B * _K
_NC, _NS = 2, 16
_NW = _NC * _NS
_BSC = 24
_BTC = _B - _BSC
_NQ = _BSC * _C // _NW
_EPW = _NQ * _K
_QROWS = _W
_RING = 3
_TBB = 8


@functools.cache
def _make_sc_gather():
    mesh = plsc.VectorSubcoreMesh(core_axis_name="c", subcore_axis_name="s")

    @functools.partial(
        pl.kernel,
        mesh=mesh,
        out_type=jax.ShapeDtypeStruct((_NW, _EPW), jnp.float32),
        compiler_params=pltpu.CompilerParams(needs_layout_passes=False),
        scratch_types=[
            pltpu.VMEM((2 * _K,), jnp.int32),
            pltpu.VMEM((2 * _K,), jnp.int32),
            pltpu.VMEM((2 * _K,), jnp.int32),
            pltpu.VMEM((_RING, _QROWS, _H), jnp.float32),
            pltpu.VMEM((_EPW,), jnp.float32),
            pltpu.SemaphoreType.DMA,
            pltpu.SemaphoreType.DMA,
            pltpu.SemaphoreType.DMA,
        ],
    )
    def _sc_gather(tab_hbm, ind_hbm, out_hbm, ind_v, w_v, h_v, chunk_v, vals_v,
                   sem0, sem1, sem2):
        wid = lax.axis_index("s") * _NC + lax.axis_index("c")
        sems = [sem0, sem1, sem2]
        g0 = wid * _NQ
        b_first = lax.shift_right_logical(g0, 3)
        row00 = g0 * _QROWS

        def issue(q, r):
            return pltpu.async_copy(
                tab_hbm.at[pl.ds(row00 + q * _QROWS, _QROWS)],
                chunk_v.at[r], sems[r])

        for r in range(_RING):
            issue(r, r)

        pltpu.sync_copy(ind_hbm.at[pl.ds(b_first * _K, 2 * _K)], ind_v)

        @pl.loop(0, 2 * _K // 16)
        def _prep(t):
            o = pl.multiple_of(t * 16, 16)
            iv = ind_v[pl.ds(o, 16)]
            h = lax.shift_right_logical(iv * 5243, 20)
            h_v[pl.ds(o, 16)] = h
            w_v[pl.ds(o, 16)] = iv - h * _W

        @pl.loop(0, _NQ, step=_RING)
        def _chunks(q0):
            for r in range(_RING):
                q = q0 + r
                pltpu.make_async_copy(
                    tab_hbm.at[pl.ds(row00, _QROWS)], chunk_v.at[r],
                    sems[r]).wait()
                g = g0 + q
                j0 = (lax.shift_right_logical(g, 3) - b_first) * _K
                for t in range(_K // 16):
                    j = pl.multiple_of(j0 + t * 16, 16)
                    vals = plsc.load_gather(
                        chunk_v.at[r], [w_v[pl.ds(j, 16)], h_v[pl.ds(j, 16)]])
                    p = pl.multiple_of(q * _K + t * 16, 16)
                    vals_v[pl.ds(p, 16)] = vals

                @pl.when(q + _RING < _NQ)
                def _():
                    issue(q + _RING, r)

        pltpu.sync_copy(vals_v, out_hbm.at[wid])

    return _sc_gather


def _tc_gather_body(ind_ref, tab_ref, out_ref):
    iota_w = lax.broadcasted_iota(jnp.int32, (_W, _K), 0)
    iota_h = lax.broadcasted_iota(jnp.int32, (_H, _K), 0)
    for bb in range(_TBB):
        iv = ind_ref[bb:bb + 1, :]
        h = lax.shift_right_logical(iv * 5243, 20)
        w = iv - h * _W
        ohf = (iota_h == jnp.broadcast_to(h, (_H, _K))).astype(jnp.float32)
        owf = (iota_w == jnp.broadcast_to(w, (_W, _K))).astype(jnp.float32)
        for c in range(_C):
            r0 = bb * _C * _W + c * _W
            slab = tab_ref[r0:r0 + _W, :]
            tmp = lax.dot_general(slab, ohf, (((1,), (0,)), ((), ())),
                                  preferred_element_type=jnp.float32)
            val = jnp.sum(owf * tmp, axis=0, keepdims=True)
            out_ref[bb:bb + 1, c * _K:(c + 1) * _K] = val


def _tc_gather(tab, ind):
    return pl.pallas_call(
        _tc_gather_body,
        grid=(_BTC // _TBB,),
        in_specs=[
            pl.BlockSpec((_TBB, _K), lambda i: (i + _BSC // _TBB, 0)),
            pl.BlockSpec((_TBB * _C * _W, _H), lambda i: (i + _BSC // _TBB, 0)),
        ],
        out_specs=pl.BlockSpec((_TBB, _C * _K), lambda i: (i, 0)),
        out_shape=jax.ShapeDtypeStruct((_BTC, _C * _K), jnp.float32),
    )(ind, tab)


def _huber(d):
    ad = jnp.abs(d)
    return jnp.where(ad < 1.0, 0.5 * ad * ad, ad - 0.5)


def _masked_mean_sum(vals, w):
    cnt = jnp.sum(w)
    s = jnp.sum(vals * w)
    return jnp.where(cnt > 0, s / jnp.maximum(cnt, 1.0), 0.0), cnt


def _tc_loss_body(psc_ref, ptc_ref, mask_ref, tb0_ref, tb1_ref, tr0_ref,
                  tr1_ref, out_ref):
    def chan(c):
        return jnp.concatenate(
            [psc_ref[:, c * _K:(c + 1) * _K], ptc_ref[:, c * _K:(c + 1) * _K]],
            axis=0)

    p = [chan(c) for c in range(_C)]
    mf = (mask_ref[...] != 0).astype(jnp.float32)
    tb0 = tb0_ref[...]
    tb1 = tb1_ref[...]
    tr0 = tr0_ref[...]
    tr1 = tr1_ref[...]

    cnt = jnp.sum(mf)

    def ce(pa, pb, tb):
        m = jnp.maximum(pa, pb)
        lse = m + jnp.log(jnp.exp(pa - m) + jnp.exp(pb - m))
        picked = jnp.where(tb == 0, pa, pb)
        s = jnp.sum((lse - picked) * mf)
        return jnp.where(cnt > 0, s / jnp.maximum(cnt, 1.0), 0.0)

    loss_bin1 = ce(p[0], p[1], tb0)
    loss_bin2 = ce(p[4], p[5], tb1)

    w1 = (tb0 != 0).astype(jnp.float32)
    ls1, c1 = _masked_mean_sum(_huber(p[2] - jnp.sin(tr0)), w1)
    lc1, _ = _masked_mean_sum(_huber(p[3] - jnp.cos(tr0)), w1)
    res1 = jnp.where(c1 > 0, ls1 + lc1, 0.0)

    w2 = (tb1 != 0).astype(jnp.float32)
    ls2, c2 = _masked_mean_sum(_huber(p[6] - jnp.sin(tr1)), w2)
    lc2, _ = _masked_mean_sum(_huber(p[7] - jnp.cos(tr1)), w2)
    res2 = jnp.where(c2 > 0, ls2 + lc2, 0.0)

    out_ref[0, 0] = loss_bin1 + loss_bin2 + res1 + res2


def _tc_loss(psc, ptc, mask, tb0, tb1, tr0, tr1):
    return pl.pallas_call(
        _tc_loss_body,
        out_shape=jax.ShapeDtypeStruct((1, 1), jnp.float32),
        out_specs=pl.BlockSpec(memory_space=pltpu.SMEM),
    )(psc, ptc, mask, tb0, tb1, tr0, tr1)


def kernel(output, mask, ind, rotbin, rotres, opt):
    tab = output.transpose(0, 1, 3, 2).reshape(_B * _C * _W, _H)
    pred_sc = _make_sc_gather()(tab, ind.reshape(-1)).reshape(_BSC, _C * _K)
    pred_tc = _tc_gather(tab, ind)
    tb0 = rotbin[:, :, 0]
    tb1 = rotbin[:, :, 1]
    tr0 = rotres[:, :, 0]
    tr1 = rotres[:, :, 1]
    return _tc_loss(pred_sc, pred_tc, mask, tb0, tb1, tr0, tr1)[0, 0]


# speedup vs baseline: 1.1028x; 1.1028x over previous
import functools

import jax
import jax.numpy as jnp
from jax import lax
from jax.experimental import pallas as pl
from jax.experimental.pallas import tpu as pltpu
from jax.experimental.pallas import tpu_sc as plsc

_B, _C, _H, _W, _K = 64, 8, 112, 200, 128
_HW = _H * _W
_N = _B * _K
_NC, _NS = 2, 16
_NW = _NC * _NS
_BSC = 32
_BTC = _B - _BSC
_NQ = _BSC * _C // _NW
_EPW = _NQ * _K
_QROWS = _W
_RING = 4
_TBB = 8


@functools.cache
def _make_sc_gather():
    mesh = plsc.VectorSubcoreMesh(core_axis_name="c", subcore_axis_name="s")

    @functools.partial(
        pl.kernel,
        mesh=mesh,
        out_type=jax.ShapeDtypeStruct((_NW, _EPW), jnp.float32),
        compiler_params=pltpu.CompilerParams(needs_layout_passes=False),
        scratch_types=[
            pltpu.VMEM((2 * _K,), jnp.int32),
            pltpu.VMEM((2 * _K,), jnp.int32),
            pltpu.VMEM((2 * _K,), jnp.int32),
            pltpu.VMEM((_RING, _QROWS, _H), jnp.float32),
            pltpu.VMEM((_EPW,), jnp.float32),
            pltpu.SemaphoreType.DMA,
            pltpu.SemaphoreType.DMA,
            pltpu.SemaphoreType.DMA,
            pltpu.SemaphoreType.DMA,
        ],
    )
    def _sc_gather(tab_hbm, ind_hbm, out_hbm, ind_v, w_v, h_v, chunk_v, vals_v,
                   sem0, sem1, sem2, sem3):
        wid = lax.axis_index("s") * _NC + lax.axis_index("c")
        sems = [sem0, sem1, sem2, sem3]
        g0 = wid * _NQ
        b_first = lax.shift_right_logical(g0, 3)
        row00 = g0 * _QROWS

        def issue(q, r):
            return pltpu.async_copy(
                tab_hbm.at[pl.ds(row00 + q * _QROWS, _QROWS)],
                chunk_v.at[r], sems[r])

        for r in range(_RING):
            issue(r, r)

        pltpu.sync_copy(ind_hbm.at[pl.ds(b_first * _K, 2 * _K)], ind_v)

        @pl.loop(0, 2 * _K // 16)
        def _prep(t):
            o = pl.multiple_of(t * 16, 16)
            iv = ind_v[pl.ds(o, 16)]
            h = lax.shift_right_logical(iv * 5243, 20)
            h_v[pl.ds(o, 16)] = h
            w_v[pl.ds(o, 16)] = iv - h * _W

        @pl.loop(0, _NQ, step=_RING)
        def _chunks(q0):
            for r in range(_RING):
                q = q0 + r
                pltpu.make_async_copy(
                    tab_hbm.at[pl.ds(row00, _QROWS)], chunk_v.at[r],
                    sems[r]).wait()
                g = g0 + q
                j0 = (lax.shift_right_logical(g, 3) - b_first) * _K
                for t in range(_K // 16):
                    j = pl.multiple_of(j0 + t * 16, 16)
                    vals = plsc.load_gather(
                        chunk_v.at[r], [w_v[pl.ds(j, 16)], h_v[pl.ds(j, 16)]])
                    p = pl.multiple_of(q * _K + t * 16, 16)
                    vals_v[pl.ds(p, 16)] = vals

                @pl.when(q + _RING < _NQ)
                def _():
                    issue(q + _RING, r)

        pltpu.sync_copy(vals_v, out_hbm.at[wid])

    return _sc_gather


def _tc_gather_body(ind_ref, tab_ref, out_ref):
    iota_w = lax.broadcasted_iota(jnp.int32, (_W, _K), 0)
    iota_h = lax.broadcasted_iota(jnp.int32, (_H, _K), 0)
    for bb in range(_TBB):
        iv = ind_ref[bb:bb + 1, :]
        h = lax.shift_right_logical(iv * 5243, 20)
        w = iv - h * _W
        ohf = (iota_h == jnp.broadcast_to(h, (_H, _K))).astype(jnp.float32)
        owf = (iota_w == jnp.broadcast_to(w, (_W, _K))).astype(jnp.float32)
        for c in range(_C):
            r0 = bb * _C * _W + c * _W
            slab = tab_ref[r0:r0 + _W, :]
            tmp = lax.dot_general(slab, ohf, (((1,), (0,)), ((), ())),
                                  preferred_element_type=jnp.float32)
            val = jnp.sum(owf * tmp, axis=0, keepdims=True)
            out_ref[bb:bb + 1, c * _K:(c + 1) * _K] = val


def _tc_gather(tab, ind):
    return pl.pallas_call(
        _tc_gather_body,
        grid=(_BTC // _TBB,),
        in_specs=[
            pl.BlockSpec((_TBB, _K), lambda i: (i + _BSC // _TBB, 0)),
            pl.BlockSpec((_TBB * _C * _W, _H), lambda i: (i + _BSC // _TBB, 0)),
        ],
        out_specs=pl.BlockSpec((_TBB, _C * _K), lambda i: (i, 0)),
        out_shape=jax.ShapeDtypeStruct((_BTC, _C * _K), jnp.float32),
    )(ind, tab)


def _huber(d):
    ad = jnp.abs(d)
    return jnp.where(ad < 1.0, 0.5 * ad * ad, ad - 0.5)


def _masked_mean_sum(vals, w):
    cnt = jnp.sum(w)
    s = jnp.sum(vals * w)
    return jnp.where(cnt > 0, s / jnp.maximum(cnt, 1.0), 0.0), cnt


def _tc_loss_body(psc_ref, ptc_ref, mask_ref, tb0_ref, tb1_ref, tr0_ref,
                  tr1_ref, out_ref):
    def chan(c):
        return jnp.concatenate(
            [psc_ref[:, c * _K:(c + 1) * _K], ptc_ref[:, c * _K:(c + 1) * _K]],
            axis=0)

    p = [chan(c) for c in range(_C)]
    mf = (mask_ref[...] != 0).astype(jnp.float32)
    tb0 = tb0_ref[...]
    tb1 = tb1_ref[...]
    tr0 = tr0_ref[...]
    tr1 = tr1_ref[...]

    cnt = jnp.sum(mf)

    def ce(pa, pb, tb):
        m = jnp.maximum(pa, pb)
        lse = m + jnp.log(jnp.exp(pa - m) + jnp.exp(pb - m))
        picked = jnp.where(tb == 0, pa, pb)
        s = jnp.sum((lse - picked) * mf)
        return jnp.where(cnt > 0, s / jnp.maximum(cnt, 1.0), 0.0)

    loss_bin1 = ce(p[0], p[1], tb0)
    loss_bin2 = ce(p[4], p[5], tb1)

    w1 = (tb0 != 0).astype(jnp.float32)
    ls1, c1 = _masked_mean_sum(_huber(p[2] - jnp.sin(tr0)), w1)
    lc1, _ = _masked_mean_sum(_huber(p[3] - jnp.cos(tr0)), w1)
    res1 = jnp.where(c1 > 0, ls1 + lc1, 0.0)

    w2 = (tb1 != 0).astype(jnp.float32)
    ls2, c2 = _masked_mean_sum(_huber(p[6] - jnp.sin(tr1)), w2)
    lc2, _ = _masked_mean_sum(_huber(p[7] - jnp.cos(tr1)), w2)
    res2 = jnp.where(c2 > 0, ls2 + lc2, 0.0)

    out_ref[0, 0] = loss_bin1 + loss_bin2 + res1 + res2


def _tc_loss(psc, ptc, mask, tb0, tb1, tr0, tr1):
    return pl.pallas_call(
        _tc_loss_body,
        out_shape=jax.ShapeDtypeStruct((1, 1), jnp.float32),
        out_specs=pl.BlockSpec(memory_space=pltpu.SMEM),
    )(psc, ptc, mask, tb0, tb1, tr0, tr1)


def kernel(output, mask, ind, rotbin, rotres, opt):
    tab = output.transpose(0, 1, 3, 2).reshape(_B * _C * _W, _H)
    pred_sc = _make_sc_gather()(tab, ind.reshape(-1)).reshape(_BSC, _C * _K)
    pred_tc = _tc_gather(tab, ind)
    tb0 = rotbin[:, :, 0]
    tb1 = rotbin[:, :, 1]
    tr0 = rotres[:, :, 0]
    tr1 = rotres[:, :, 1]
    return _tc_loss(pred_sc, pred_tc, mask, tb0, tb1, tr0, tr1)[0, 0]
